# baseline (speedup 1.0000x reference)
import functools

import jax
import jax.numpy as jnp
import numpy as np
from jax.experimental import pallas as pl
from jax.experimental.pallas import tpu as pltpu

B, S, V = 8, 8, 100000
EPS = 1e-12
KU0, KU1 = np.uint32(507451445), np.uint32(1853169794)
KC0, KC1 = np.uint32(1948878966), np.uint32(4237131848)
TINY = np.float32(np.finfo(np.float32).tiny)
U_SCALE = np.float32(0.999 - EPS)
U_SHIFT = np.float32(EPS)


def _threefry_bits(k0, k1, counts):
    k0 = jnp.uint32(k0)
    k1 = jnp.uint32(k1)
    ks2 = k0 ^ k1 ^ jnp.uint32(0x1BD11BDA)
    rots = ((13, 15, 26, 6), (17, 29, 16, 24))
    ks = (k0, k1, ks2)
    x0 = jnp.full_like(counts, k0)
    x1 = counts + k1
    for i in range(5):
        for r in rots[i % 2]:
            x0 = x0 + x1
            x1 = (x1 << np.uint32(r)) | (x1 >> np.uint32(32 - r))
            x1 = x1 ^ x0
        x0 = x0 + ks[(i + 1) % 3]
        x1 = x1 + ks[(i + 2) % 3] + jnp.uint32(i + 1)
    return x0 ^ x1


def _unit_float(bits):
    fb = (bits >> np.uint32(9)) | np.uint32(0x3F800000)
    return jax.lax.bitcast_convert_type(fb, jnp.float32) - np.float32(1.0)


def _stgs_kernel(x_ref, y1_ref, y2_ref, diff_ref, ids_s, gath_s):
    r = pl.program_id(0)
    xv = x_ref[0]
    s_iota = jax.lax.broadcasted_iota(jnp.uint32, (S, V), 0)
    v_iota = jax.lax.broadcasted_iota(jnp.uint32, (S, V), 1)
    base = jnp.uint32(r) * jnp.uint32(S * V)
    counts = base + s_iota * jnp.uint32(V) + v_iota

    u = _unit_float(_threefry_bits(KU0, KU1, counts))
    u = u * U_SCALE + U_SHIFT
    g1 = -jnp.log(-jnp.log(u))
    logits = xv + g1

    m = jnp.max(logits, axis=1, keepdims=True)
    e = jnp.exp(logits - m)
    ssum = jnp.sum(e, axis=1, keepdims=True)
    y = e / ssum
    y1_ref[0] = y
    y2_ref[0] = y

    uc = _unit_float(_threefry_bits(KC0, KC1, counts))
    uc = jnp.maximum(TINY, uc + TINY)
    g2 = -jnp.log(-jnp.log(uc))
    t = (logits - m) + g2
    tmax = jnp.max(t, axis=1, keepdims=True)
    vi = jax.lax.broadcasted_iota(jnp.int32, (S, V), 1)
    big = jnp.int32(2**31 - 1)
    idx = jnp.min(jnp.where(t == tmax, vi, big), axis=1, keepdims=True)
    gath = jnp.sum(jnp.where(vi == idx, y, 0.0), axis=1, keepdims=True)

    lane = jax.lax.broadcasted_iota(jnp.int32, (S, B), 1)
    col = lane == r
    ids_s[...] = jnp.where(col, idx.astype(jnp.float32), ids_s[...])
    gath_s[...] = jnp.where(col, gath, gath_s[...])

    ids_m = ids_s[...].T
    g_m = gath_s[...].T
    diff_ref[...] = (ids_m[None, :, :] - g_m[:, :, None]) + g_m[:, :, None]


@functools.partial(jax.jit, static_argnames=())
def _stgs(x):
    grid = (B,)
    y1, y2, diff = pl.pallas_call(
        _stgs_kernel,
        grid=grid,
        in_specs=[pl.BlockSpec((1, S, V), lambda r: (r, 0, 0))],
        out_specs=[
            pl.BlockSpec((1, S, V), lambda r: (r, 0, 0)),
            pl.BlockSpec((1, S, V), lambda r: (r, 0, 0)),
            pl.BlockSpec((B, S, S), lambda r: (0, 0, 0)),
        ],
        out_shape=[
            jax.ShapeDtypeStruct((B, S, V), jnp.float32),
            jax.ShapeDtypeStruct((B, S, V), jnp.float32),
            jax.ShapeDtypeStruct((B, S, S), jnp.float32),
        ],
        scratch_shapes=[
            pltpu.VMEM((S, B), jnp.float32),
            pltpu.VMEM((S, B), jnp.float32),
        ],
    )(x)
    return y1, y2, diff


def kernel(x):
    y1, y2, diff = _stgs(x)
    eff_temperature = jnp.array([1.0], dtype=jnp.float32)
    return (diff, y1, eff_temperature, y2)
